# baseline (speedup 1.0000x reference)
import functools

import jax
import jax.numpy as jnp
from jax import lax
from jax.experimental import pallas as pl
from jax.experimental.pallas import tpu as pltpu
from jax.experimental.pallas import tpu_sc as plsc

N_ROWS = 1_000_000
D_COLS = 64
Q = 16384
D_SEL = 32

NUM_CORES = 2
NUM_SUBCORES = 16
NW = NUM_CORES * NUM_SUBCORES
BPW = Q // NW
CHUNK = 128
NCHUNK = BPW // CHUNK
LANES = 16


def _query_body(table_hbm, rows_hbm, cols_hbm, out_hbm,
                idx_v, cols_v, rows_v, out_v, sem):
    wid = lax.axis_index("s") * NUM_CORES + lax.axis_index("c")
    base = wid * BPW

    for j in range(NCHUNK):
        pltpu.sync_copy(rows_hbm.at[pl.ds(base + j * CHUNK, CHUNK)],
                        idx_v.at[j])
    pltpu.sync_copy(cols_hbm, cols_v)

    copies = []
    for j in range(NCHUNK):
        copies.append(pltpu.async_copy(
            table_hbm.at[idx_v.at[j]],
            rows_v.at[pl.ds(j * CHUNK, CHUNK)],
            sem))
    for c in copies:
        c.wait()

    c_lo = cols_v[pl.ds(0, LANES)]
    c_hi = cols_v[pl.ds(LANES, LANES)]

    def body(r, carry):
        ridx = jnp.full((LANES,), r, dtype=jnp.int32)
        out_v[r, pl.ds(0, LANES)] = plsc.load_gather(rows_v, [ridx, c_lo])
        out_v[r, pl.ds(LANES, LANES)] = plsc.load_gather(rows_v, [ridx, c_hi])
        return carry

    lax.fori_loop(0, BPW, body, 0, unroll=4)

    pltpu.sync_copy(out_v, out_hbm.at[pl.ds(base, BPW)])


@jax.jit
def kernel(table, rows, cols):
    mesh = plsc.VectorSubcoreMesh(
        core_axis_name="c", subcore_axis_name="s",
        num_cores=NUM_CORES, num_subcores=NUM_SUBCORES)
    run = pl.kernel(
        _query_body,
        out_type=jax.ShapeDtypeStruct((Q, D_SEL), jnp.float32),
        mesh=mesh,
        scratch_types=[
            pltpu.VMEM((NCHUNK, CHUNK), jnp.int32),
            pltpu.VMEM((D_SEL,), jnp.int32),
            pltpu.VMEM((BPW, D_COLS), jnp.float32),
            pltpu.VMEM((BPW, D_SEL), jnp.float32),
            pltpu.SemaphoreType.DMA,
        ],
        compiler_params=pltpu.CompilerParams(
            needs_layout_passes=False, use_tc_tiling_on_sc=False),
    )
    return run(table, rows.astype(jnp.int32), cols.astype(jnp.int32))
